# Initial kernel scaffold; baseline (speedup 1.0000x reference)
import jax
import jax.numpy as jnp
from jax.experimental import pallas as pl


def kernel(xyz, knn, W1, b1, W2, b2, W3a, b3a, W3b, b3b):
    raise NotImplementedError("write your pallas kernel here")


# speedup vs baseline: 4.0349x; 4.0349x over previous
import functools

import jax
import jax.numpy as jnp
from jax import lax
from jax.experimental import pallas as pl
from jax.experimental.pallas import tpu as pltpu
from jax.experimental.pallas import tpu_sc as plsc

N = 50000
K = 16
HALF = 64
DIM = 128
OUT_DIM = 64

NC = 2
NS = 16
NW = NC * NS
PW = 1568
NPAD = NW * PW
CH = 56
NB = 1000


def _k1_body(xyz_ref, w1_ref, b1_ref, w2_ref, b2_ref, t_ref):
    x = xyz_ref[...]
    w1 = w1_ref[...]
    f0 = (x[:, 0:1] * w1[0:1, :] + x[:, 1:2] * w1[1:2, :]
          + x[:, 2:3] * w1[2:3, :] + b1_ref[...])
    f1 = jnp.dot(f0, w2_ref[...], preferred_element_type=jnp.float32) + b2_ref[...]
    t_ref[...] = jnp.concatenate([f0, f1], axis=1)


def _k1(xyz, W1, b1, W2, b2):
    return pl.pallas_call(
        _k1_body,
        grid=(N // NB,),
        in_specs=[
            pl.BlockSpec((NB, 3), lambda i: (i, 0)),
            pl.BlockSpec((3, HALF), lambda i: (0, 0)),
            pl.BlockSpec((1, HALF), lambda i: (0, 0)),
            pl.BlockSpec((HALF, HALF), lambda i: (0, 0)),
            pl.BlockSpec((1, HALF), lambda i: (0, 0)),
        ],
        out_specs=pl.BlockSpec((NB, DIM), lambda i: (i, 0)),
        out_shape=jax.ShapeDtypeStruct((N, DIM), jnp.float32),
    )(xyz, W1, b1, W2, b2)


def _k2_body(knn_hbm, t_hbm, gmax_hbm, tg_hbm, idx_v, rows_v, gmax_v, sem):
    wid = lax.axis_index("s") * NC + lax.axis_index("c")
    w_base = wid * PW

    def chunk(ch, _):
        node_base = w_base + ch * CH
        e_base = node_base * K
        pltpu.sync_copy(knn_hbm.at[pl.ds(e_base, CH * K)], idx_v)
        pltpu.async_copy(t_hbm.at[idx_v], rows_v, sem).wait()

        def node_max(n, _):
            r0 = n * K
            for col in range(HALF // 16):
                cs = col * 16
                acc = rows_v[r0, pl.ds(cs, 16)]
                for r in range(1, K):
                    acc = jnp.maximum(acc, rows_v[r0 + r, pl.ds(cs, 16)])
                gmax_v[n, pl.ds(cs, 16)] = acc
            return ()

        lax.fori_loop(0, CH, node_max, (), unroll=False)
        pltpu.sync_copy(rows_v, tg_hbm.at[pl.ds(e_base, CH * K)])
        pltpu.sync_copy(gmax_v, gmax_hbm.at[pl.ds(node_base, CH)])
        return ()

    lax.fori_loop(0, PW // CH, chunk, (), unroll=False)


def _k2(knn_flat_pad, t):
    mesh = plsc.VectorSubcoreMesh(core_axis_name="c", subcore_axis_name="s",
                                  num_cores=NC, num_subcores=NS)
    fn = functools.partial(
        pl.kernel,
        out_type=[
            jax.ShapeDtypeStruct((NPAD, HALF), jnp.float32),
            jax.ShapeDtypeStruct((NPAD * K, DIM), jnp.float32),
        ],
        mesh=mesh,
        scratch_types=[
            pltpu.VMEM((CH * K,), jnp.int32),
            pltpu.VMEM((CH * K, DIM), jnp.float32),
            pltpu.VMEM((CH, HALF), jnp.float32),
            pltpu.SemaphoreType.DMA,
        ],
    )(_k2_body)
    return fn(knn_flat_pad, t)


_ERF_A1 = 0.254829592
_ERF_A2 = -0.284496736
_ERF_A3 = 1.421413741
_ERF_A4 = -1.453152027
_ERF_A5 = 1.061405429
_ERF_P = 0.3275911


def _gelu_exact(x):
    z = x * 0.7071067811865476
    az = jnp.abs(z)
    t = 1.0 / (1.0 + _ERF_P * az)
    poly = t * (_ERF_A1 + t * (_ERF_A2 + t * (_ERF_A3 + t * (_ERF_A4 + t * _ERF_A5))))
    erf = jnp.sign(z) * (1.0 - poly * jnp.exp(-az * az))
    return 0.5 * x * (1.0 + erf)


def _k3_body(tg_ref, t_ref, gmax_ref, w3hi_ref, wnode_ref, wcat_ref,
             b3a_ref, w3b_ref, b3b_ref, out_ref):
    c = (jnp.dot(gmax_ref[...], w3hi_ref[...], preferred_element_type=jnp.float32)
         - jnp.dot(t_ref[...], wnode_ref[...], preferred_element_type=jnp.float32)
         + b3a_ref[...])
    h = jnp.dot(tg_ref[...], wcat_ref[...], preferred_element_type=jnp.float32)
    h = h.reshape(NB, K, DIM) + c[:, None, :]
    h = _gelu_exact(h).reshape(NB * K, DIM)
    out = jnp.dot(h, w3b_ref[...], preferred_element_type=jnp.float32) + b3b_ref[...]
    out_ref[...] = out.reshape(NB, K, OUT_DIM)


def _k3(tg, t, gmax, W3a_hi, Wnode, Wcat, b3a, W3b, b3b):
    return pl.pallas_call(
        _k3_body,
        grid=(N // NB,),
        in_specs=[
            pl.BlockSpec((NB * K, DIM), lambda i: (i, 0)),
            pl.BlockSpec((NB, DIM), lambda i: (i, 0)),
            pl.BlockSpec((NB, HALF), lambda i: (i, 0)),
            pl.BlockSpec((HALF, DIM), lambda i: (0, 0)),
            pl.BlockSpec((DIM, DIM), lambda i: (0, 0)),
            pl.BlockSpec((DIM, DIM), lambda i: (0, 0)),
            pl.BlockSpec((1, DIM), lambda i: (0, 0)),
            pl.BlockSpec((DIM, OUT_DIM), lambda i: (0, 0)),
            pl.BlockSpec((1, OUT_DIM), lambda i: (0, 0)),
        ],
        out_specs=pl.BlockSpec((NB, K, OUT_DIM), lambda i: (i, 0, 0)),
        out_shape=jax.ShapeDtypeStruct((N, K, OUT_DIM), jnp.float32),
    )(tg, t, gmax, W3a_hi, Wnode, Wcat, b3a, W3b, b3b)


def kernel(xyz, knn, W1, b1, W2, b2, W3a, b3a, W3b, b3b):
    W3a_lo = W3a[:HALF, :]
    W3a_hi = W3a[HALF:, :]
    Wnode = jnp.concatenate([W3a_hi, W3a_lo], axis=0)
    Wcat = jnp.concatenate([jnp.zeros_like(W3a_lo), W3a_lo], axis=0)
    b1r = b1.reshape(1, HALF)
    b2r = b2.reshape(1, HALF)
    b3ar = b3a.reshape(1, DIM)
    b3br = b3b.reshape(1, OUT_DIM)

    t = _k1(xyz, W1, b1r, W2, b2r)

    knn_flat = knn.reshape(-1).astype(jnp.int32)
    knn_flat_pad = jnp.pad(knn_flat, (0, (NPAD - N) * K))
    gmax_pad, tg_pad = _k2(knn_flat_pad, t)

    return _k3(tg_pad, t, gmax_pad, W3a_hi, Wnode, Wcat, b3ar, W3b, b3br)


# speedup vs baseline: 4.6190x; 1.1447x over previous
import functools

import jax
import jax.numpy as jnp
from jax import lax
from jax.experimental import pallas as pl
from jax.experimental.pallas import tpu as pltpu
from jax.experimental.pallas import tpu_sc as plsc

N = 50000
K = 16
HALF = 64
DIM = 128
OUT_DIM = 64

NC = 2
NS = 16
NW = NC * NS
PW = 1568
NPAD = NW * PW
CH = 56
NB = 1000


def _k1_body(xyz_ref, w1_ref, b1_ref, w2_ref, b2_ref, t_ref):
    x = xyz_ref[...]
    w1 = w1_ref[...]
    f0 = (x[:, 0:1] * w1[0:1, :] + x[:, 1:2] * w1[1:2, :]
          + x[:, 2:3] * w1[2:3, :] + b1_ref[...])
    f1 = jnp.dot(f0, w2_ref[...], preferred_element_type=jnp.float32) + b2_ref[...]
    t_ref[...] = jnp.concatenate([f0, f1], axis=1)


def _k1(xyz, W1, b1, W2, b2):
    return pl.pallas_call(
        _k1_body,
        grid=(N // NB,),
        in_specs=[
            pl.BlockSpec((NB, 3), lambda i: (i, 0)),
            pl.BlockSpec((3, HALF), lambda i: (0, 0)),
            pl.BlockSpec((1, HALF), lambda i: (0, 0)),
            pl.BlockSpec((HALF, HALF), lambda i: (0, 0)),
            pl.BlockSpec((1, HALF), lambda i: (0, 0)),
        ],
        out_specs=pl.BlockSpec((NB, DIM), lambda i: (i, 0)),
        out_shape=jax.ShapeDtypeStruct((N, DIM), jnp.float32),
    )(xyz, W1, b1, W2, b2)


def _k2_body(knn_hbm, t_hbm, gmax_hbm, tg_hbm, idx_v, rows_v, gmax_v, sem):
    wid = lax.axis_index("s") * NC + lax.axis_index("c")
    w_base = wid * PW

    def chunk(ch, _):
        node_base = w_base + ch * CH
        e_base = node_base * K
        pltpu.sync_copy(knn_hbm.at[pl.ds(e_base, CH * K)], idx_v)
        pltpu.async_copy(t_hbm.at[idx_v], rows_v, sem).wait()

        def node_max(n, _):
            r0 = n * K
            for col in range(HALF // 16):
                cs = col * 16
                acc = rows_v[r0, pl.ds(cs, 16)]
                for r in range(1, K):
                    acc = jnp.maximum(acc, rows_v[r0 + r, pl.ds(cs, 16)])
                gmax_v[n, pl.ds(cs, 16)] = acc
            return ()

        lax.fori_loop(0, CH, node_max, (), unroll=False)
        pltpu.sync_copy(rows_v, tg_hbm.at[pl.ds(e_base, CH * K)])
        pltpu.sync_copy(gmax_v, gmax_hbm.at[pl.ds(node_base, CH)])
        return ()

    lax.fori_loop(0, PW // CH, chunk, (), unroll=False)


def _k2(knn_flat_pad, t):
    mesh = plsc.VectorSubcoreMesh(core_axis_name="c", subcore_axis_name="s",
                                  num_cores=NC, num_subcores=NS)
    fn = functools.partial(
        pl.kernel,
        out_type=[
            jax.ShapeDtypeStruct((NPAD, HALF), jnp.float32),
            jax.ShapeDtypeStruct((NPAD * K, DIM), jnp.float32),
        ],
        mesh=mesh,
        scratch_types=[
            pltpu.VMEM((CH * K,), jnp.int32),
            pltpu.VMEM((CH * K, DIM), jnp.float32),
            pltpu.VMEM((CH, HALF), jnp.float32),
            pltpu.SemaphoreType.DMA,
        ],
    )(_k2_body)
    return fn(knn_flat_pad, t)


def _gelu_exact(x):
    return 0.5 * x * (1.0 + lax.erf(x * 0.7071067811865476))


def _k3_body(tg_ref, t_ref, gmax_ref, w3hi_ref, wnode_ref, wcat_ref,
             b3a_ref, w3b_ref, b3b_ref, out_ref):
    c = (jnp.dot(gmax_ref[...], w3hi_ref[...], preferred_element_type=jnp.float32)
         - jnp.dot(t_ref[...], wnode_ref[...], preferred_element_type=jnp.float32)
         + b3a_ref[...])
    h = jnp.dot(tg_ref[...], wcat_ref[...], preferred_element_type=jnp.float32)
    h = h.reshape(NB, K, DIM) + c[:, None, :]
    h = _gelu_exact(h).reshape(NB * K, DIM)
    out = jnp.dot(h, w3b_ref[...], preferred_element_type=jnp.float32) + b3b_ref[...]
    out_ref[...] = out.reshape(NB, K, OUT_DIM)


def _k3(tg, t, gmax, W3a_hi, Wnode, Wcat, b3a, W3b, b3b):
    return pl.pallas_call(
        _k3_body,
        grid=(N // NB,),
        in_specs=[
            pl.BlockSpec((NB * K, DIM), lambda i: (i, 0)),
            pl.BlockSpec((NB, DIM), lambda i: (i, 0)),
            pl.BlockSpec((NB, HALF), lambda i: (i, 0)),
            pl.BlockSpec((HALF, DIM), lambda i: (0, 0)),
            pl.BlockSpec((DIM, DIM), lambda i: (0, 0)),
            pl.BlockSpec((DIM, DIM), lambda i: (0, 0)),
            pl.BlockSpec((1, DIM), lambda i: (0, 0)),
            pl.BlockSpec((DIM, OUT_DIM), lambda i: (0, 0)),
            pl.BlockSpec((1, OUT_DIM), lambda i: (0, 0)),
        ],
        out_specs=pl.BlockSpec((NB, K, OUT_DIM), lambda i: (i, 0, 0)),
        out_shape=jax.ShapeDtypeStruct((N, K, OUT_DIM), jnp.float32),
    )(tg, t, gmax, W3a_hi, Wnode, Wcat, b3a, W3b, b3b)


def kernel(xyz, knn, W1, b1, W2, b2, W3a, b3a, W3b, b3b):
    W3a_lo = W3a[:HALF, :]
    W3a_hi = W3a[HALF:, :]
    Wnode = jnp.concatenate([W3a_hi, W3a_lo], axis=0)
    Wcat = jnp.concatenate([jnp.zeros_like(W3a_lo), W3a_lo], axis=0)
    b1r = b1.reshape(1, HALF)
    b2r = b2.reshape(1, HALF)
    b3ar = b3a.reshape(1, DIM)
    b3br = b3b.reshape(1, OUT_DIM)

    t = _k1(xyz, W1, b1r, W2, b2r)

    knn_flat = knn.reshape(-1).astype(jnp.int32)
    knn_flat_pad = jnp.pad(knn_flat, (0, (NPAD - N) * K))
    gmax_pad, tg_pad = _k2(knn_flat_pad, t)

    return _k3(tg_pad, t, gmax_pad, W3a_hi, Wnode, Wcat, b3ar, W3b, b3br)


# speedup vs baseline: 4.9867x; 1.0796x over previous
import functools

import jax
import jax.numpy as jnp
from jax import lax
from jax.experimental import pallas as pl
from jax.experimental.pallas import tpu as pltpu
from jax.experimental.pallas import tpu_sc as plsc

N = 50000
K = 16
HALF = 64
DIM = 128
OUT_DIM = 64

NC = 2
NS = 16
NW = NC * NS
PW = 1568
NPAD = NW * PW
CH = 16
CHE = CH * K
NCHUNK = PW // CH
NB = 1000


def _k1_body(xyz_ref, w1_ref, b1_ref, w2_ref, b2_ref, t_ref):
    x = xyz_ref[...]
    w1 = w1_ref[...]
    f0 = (x[:, 0:1] * w1[0:1, :] + x[:, 1:2] * w1[1:2, :]
          + x[:, 2:3] * w1[2:3, :] + b1_ref[...])
    f1 = jnp.dot(f0, w2_ref[...], preferred_element_type=jnp.float32) + b2_ref[...]
    t_ref[...] = jnp.concatenate([f0, f1], axis=1)


def _k1(xyz, W1, b1, W2, b2):
    return pl.pallas_call(
        _k1_body,
        grid=(N // NB,),
        in_specs=[
            pl.BlockSpec((NB, 3), lambda i: (i, 0)),
            pl.BlockSpec((3, HALF), lambda i: (0, 0)),
            pl.BlockSpec((1, HALF), lambda i: (0, 0)),
            pl.BlockSpec((HALF, HALF), lambda i: (0, 0)),
            pl.BlockSpec((1, HALF), lambda i: (0, 0)),
        ],
        out_specs=pl.BlockSpec((NB, DIM), lambda i: (i, 0)),
        out_shape=jax.ShapeDtypeStruct((N, DIM), jnp.float32),
    )(xyz, W1, b1, W2, b2)


def _k2_body(knn_hbm, t_hbm, gmax_hbm, tg_hbm, idx_all,
             rows0, rows1, gm0, gm1, gs0, gs1, ss0, ss1, ms0, ms1):
    wid = lax.axis_index("s") * NC + lax.axis_index("c")
    w_base = wid * PW
    rows = (rows0, rows1)
    gm = (gm0, gm1)
    gsem = (gs0, gs1)
    ssem = (ss0, ss1)
    msem = (ms0, ms1)

    pltpu.sync_copy(knn_hbm.at[pl.ds(w_base * K, PW * K)], idx_all)

    def start_gather(ch, b):
        pltpu.async_copy(t_hbm.at[idx_all.at[pl.ds(ch * CHE, CHE)]],
                         rows[b], gsem[b])

    def drain_stores(b):
        pltpu.make_async_copy(rows[b], tg_hbm.at[pl.ds(0, CHE)], ssem[b]).wait()
        pltpu.make_async_copy(gm[b], gmax_hbm.at[pl.ds(0, CH)], msem[b]).wait()

    def process(ch, b):
        node_base = w_base + ch * CH

        def node_max(n, _):
            r0 = n * K
            for col in range(HALF // 16):
                cs = col * 16
                acc = rows[b][r0, pl.ds(cs, 16)]
                for r in range(1, K):
                    acc = jnp.maximum(acc, rows[b][r0 + r, pl.ds(cs, 16)])
                gm[b][n, pl.ds(cs, 16)] = acc
            return ()

        lax.fori_loop(0, CH, node_max, (), unroll=False)
        pltpu.async_copy(rows[b], tg_hbm.at[pl.ds(node_base * K, CHE)], ssem[b])
        pltpu.async_copy(gm[b], gmax_hbm.at[pl.ds(node_base, CH)], msem[b])

    start_gather(0, 0)
    pltpu.make_async_copy(t_hbm.at[idx_all.at[pl.ds(0, CHE)]], rows[0],
                          gsem[0]).wait()
    start_gather(1, 1)
    process(0, 0)
    pltpu.make_async_copy(t_hbm.at[idx_all.at[pl.ds(0, CHE)]], rows[1],
                          gsem[1]).wait()
    drain_stores(0)
    start_gather(2, 0)
    process(1, 1)

    def two_chunks(i, _):
        ch = 2 * i
        for b in range(2):
            pltpu.make_async_copy(t_hbm.at[idx_all.at[pl.ds(0, CHE)]],
                                  rows[b], gsem[b]).wait()
            drain_stores(1 - b)

            @pl.when(ch + b + 1 < NCHUNK)
            def _():
                start_gather(ch + b + 1, 1 - b)

            process(ch + b, b)
        return ()

    lax.fori_loop(1, NCHUNK // 2, two_chunks, (), unroll=False)
    drain_stores(1)


def _k2(knn_flat_pad, t):
    mesh = plsc.VectorSubcoreMesh(core_axis_name="c", subcore_axis_name="s",
                                  num_cores=NC, num_subcores=NS)
    fn = functools.partial(
        pl.kernel,
        out_type=[
            jax.ShapeDtypeStruct((NPAD, HALF), jnp.float32),
            jax.ShapeDtypeStruct((NPAD * K, DIM), jnp.float32),
        ],
        mesh=mesh,
        scratch_types=[
            pltpu.VMEM((PW * K,), jnp.int32),
            pltpu.VMEM((CHE, DIM), jnp.float32),
            pltpu.VMEM((CHE, DIM), jnp.float32),
            pltpu.VMEM((CH, HALF), jnp.float32),
            pltpu.VMEM((CH, HALF), jnp.float32),
            pltpu.SemaphoreType.DMA,
            pltpu.SemaphoreType.DMA,
            pltpu.SemaphoreType.DMA,
            pltpu.SemaphoreType.DMA,
            pltpu.SemaphoreType.DMA,
            pltpu.SemaphoreType.DMA,
        ],
    )(_k2_body)
    return fn(knn_flat_pad, t)


def _gelu_exact(x):
    return 0.5 * x * (1.0 + lax.erf(x * 0.7071067811865476))


def _k3_body(tg_ref, t_ref, gmax_ref, w3hi_ref, wnode_ref, wcat_ref,
             b3a_ref, w3b_ref, b3b_ref, out_ref):
    c = (jnp.dot(gmax_ref[...], w3hi_ref[...], preferred_element_type=jnp.float32)
         - jnp.dot(t_ref[...], wnode_ref[...], preferred_element_type=jnp.float32)
         + b3a_ref[...])
    h = jnp.dot(tg_ref[...], wcat_ref[...], preferred_element_type=jnp.float32)
    h = h.reshape(NB, K, DIM) + c[:, None, :]
    h = _gelu_exact(h).reshape(NB * K, DIM)
    out = jnp.dot(h, w3b_ref[...], preferred_element_type=jnp.float32) + b3b_ref[...]
    out_ref[...] = out.reshape(NB, K, OUT_DIM)


def _k3(tg, t, gmax, W3a_hi, Wnode, Wcat, b3a, W3b, b3b):
    return pl.pallas_call(
        _k3_body,
        grid=(N // NB,),
        in_specs=[
            pl.BlockSpec((NB * K, DIM), lambda i: (i, 0)),
            pl.BlockSpec((NB, DIM), lambda i: (i, 0)),
            pl.BlockSpec((NB, HALF), lambda i: (i, 0)),
            pl.BlockSpec((HALF, DIM), lambda i: (0, 0)),
            pl.BlockSpec((DIM, DIM), lambda i: (0, 0)),
            pl.BlockSpec((DIM, DIM), lambda i: (0, 0)),
            pl.BlockSpec((1, DIM), lambda i: (0, 0)),
            pl.BlockSpec((DIM, OUT_DIM), lambda i: (0, 0)),
            pl.BlockSpec((1, OUT_DIM), lambda i: (0, 0)),
        ],
        out_specs=pl.BlockSpec((NB, K, OUT_DIM), lambda i: (i, 0, 0)),
        out_shape=jax.ShapeDtypeStruct((N, K, OUT_DIM), jnp.float32),
    )(tg, t, gmax, W3a_hi, Wnode, Wcat, b3a, W3b, b3b)


def kernel(xyz, knn, W1, b1, W2, b2, W3a, b3a, W3b, b3b):
    W3a_lo = W3a[:HALF, :]
    W3a_hi = W3a[HALF:, :]
    Wnode = jnp.concatenate([W3a_hi, W3a_lo], axis=0)
    Wcat = jnp.concatenate([jnp.zeros_like(W3a_lo), W3a_lo], axis=0)
    b1r = b1.reshape(1, HALF)
    b2r = b2.reshape(1, HALF)
    b3ar = b3a.reshape(1, DIM)
    b3br = b3b.reshape(1, OUT_DIM)

    t = _k1(xyz, W1, b1r, W2, b2r)

    knn_flat = knn.reshape(-1).astype(jnp.int32)
    knn_flat_pad = jnp.pad(knn_flat, (0, (NPAD - N) * K))
    gmax_pad, tg_pad = _k2(knn_flat_pad, t)

    return _k3(tg_pad, t, gmax_pad, W3a_hi, Wnode, Wcat, b3ar, W3b, b3br)


# speedup vs baseline: 5.0523x; 1.0131x over previous
import functools

import jax
import jax.numpy as jnp
from jax import lax
from jax.experimental import pallas as pl
from jax.experimental.pallas import tpu as pltpu
from jax.experimental.pallas import tpu_sc as plsc

N = 50000
K = 16
HALF = 64
DIM = 128
OUT_DIM = 64

NC = 2
NS = 16
NW = NC * NS
PW = 1568
NPAD = NW * PW
CH = 16
CHE = CH * K
NCHUNK = PW // CH
NB = 1000


def _k1_body(xyz_ref, w1_ref, b1_ref, w2_ref, b2_ref, t_ref):
    x = xyz_ref[...]
    w1 = w1_ref[...]
    f0 = (x[:, 0:1] * w1[0:1, :] + x[:, 1:2] * w1[1:2, :]
          + x[:, 2:3] * w1[2:3, :] + b1_ref[...])
    f1 = jnp.dot(f0, w2_ref[...], preferred_element_type=jnp.float32) + b2_ref[...]
    t_ref[...] = jnp.concatenate([f0, f1], axis=1)


def _k1(xyz, W1, b1, W2, b2):
    return pl.pallas_call(
        _k1_body,
        grid=(N // NB,),
        in_specs=[
            pl.BlockSpec((NB, 3), lambda i: (i, 0)),
            pl.BlockSpec((3, HALF), lambda i: (0, 0)),
            pl.BlockSpec((1, HALF), lambda i: (0, 0)),
            pl.BlockSpec((HALF, HALF), lambda i: (0, 0)),
            pl.BlockSpec((1, HALF), lambda i: (0, 0)),
        ],
        out_specs=pl.BlockSpec((NB, DIM), lambda i: (i, 0)),
        out_shape=jax.ShapeDtypeStruct((N, DIM), jnp.float32),
    )(xyz, W1, b1, W2, b2)


def _k2_body(knn_hbm, t_hbm, tg_hbm, idx_all,
             rows0, rows1, gs0, gs1, ss0, ss1):
    wid = lax.axis_index("s") * NC + lax.axis_index("c")
    w_base = wid * PW
    rows = (rows0, rows1)
    gsem = (gs0, gs1)
    ssem = (ss0, ss1)

    pltpu.sync_copy(knn_hbm.at[pl.ds(w_base * K, PW * K)], idx_all)

    def start_gather(ch, b):
        pltpu.async_copy(t_hbm.at[idx_all.at[pl.ds(ch * CHE, CHE)]],
                         rows[b], gsem[b])

    def wait_gather(b):
        pltpu.make_async_copy(t_hbm.at[idx_all.at[pl.ds(0, CHE)]], rows[b],
                              gsem[b]).wait()

    def drain_store(b):
        pltpu.make_async_copy(rows[b], tg_hbm.at[pl.ds(0, CHE)], ssem[b]).wait()

    def store(ch, b):
        e_base = (w_base + ch * CH) * K
        pltpu.async_copy(rows[b], tg_hbm.at[pl.ds(e_base, CHE)], ssem[b])

    start_gather(0, 0)
    wait_gather(0)
    start_gather(1, 1)
    store(0, 0)
    wait_gather(1)
    drain_store(0)
    start_gather(2, 0)
    store(1, 1)

    def two_chunks(i, _):
        ch = 2 * i
        for b in range(2):
            wait_gather(b)
            drain_store(1 - b)

            @pl.when(ch + b + 1 < NCHUNK)
            def _():
                start_gather(ch + b + 1, 1 - b)

            store(ch + b, b)
        return ()

    lax.fori_loop(1, NCHUNK // 2, two_chunks, (), unroll=False)
    drain_store(1)


def _k2(knn_flat_pad, t):
    mesh = plsc.VectorSubcoreMesh(core_axis_name="c", subcore_axis_name="s",
                                  num_cores=NC, num_subcores=NS)
    fn = functools.partial(
        pl.kernel,
        out_type=jax.ShapeDtypeStruct((NPAD * K, DIM), jnp.float32),
        mesh=mesh,
        scratch_types=[
            pltpu.VMEM((PW * K,), jnp.int32),
            pltpu.VMEM((CHE, DIM), jnp.float32),
            pltpu.VMEM((CHE, DIM), jnp.float32),
            pltpu.SemaphoreType.DMA,
            pltpu.SemaphoreType.DMA,
            pltpu.SemaphoreType.DMA,
            pltpu.SemaphoreType.DMA,
        ],
    )(_k2_body)
    return fn(knn_flat_pad, t)


def _gelu_exact(x):
    return 0.5 * x * (1.0 + lax.erf(x * 0.7071067811865476))


def _k3_body(tg_ref, t_ref, w3hi_ref, wnode_ref, wcat_ref,
             b3a_ref, w3b_ref, b3b_ref, out_ref):
    tg3 = tg_ref[...].reshape(NB, K, DIM)
    gmax = jnp.max(tg3[:, :, :HALF], axis=1)
    c = (jnp.dot(gmax, w3hi_ref[...], preferred_element_type=jnp.float32)
         - jnp.dot(t_ref[...], wnode_ref[...], preferred_element_type=jnp.float32)
         + b3a_ref[...])
    h = jnp.dot(tg_ref[...], wcat_ref[...], preferred_element_type=jnp.float32)
    h = h.reshape(NB, K, DIM) + c[:, None, :]
    h = _gelu_exact(h).reshape(NB * K, DIM)
    out = jnp.dot(h, w3b_ref[...], preferred_element_type=jnp.float32) + b3b_ref[...]
    out_ref[...] = out.reshape(NB, K, OUT_DIM)


def _k3(tg, t, W3a_hi, Wnode, Wcat, b3a, W3b, b3b):
    return pl.pallas_call(
        _k3_body,
        grid=(N // NB,),
        in_specs=[
            pl.BlockSpec((NB * K, DIM), lambda i: (i, 0)),
            pl.BlockSpec((NB, DIM), lambda i: (i, 0)),
            pl.BlockSpec((HALF, DIM), lambda i: (0, 0)),
            pl.BlockSpec((DIM, DIM), lambda i: (0, 0)),
            pl.BlockSpec((DIM, DIM), lambda i: (0, 0)),
            pl.BlockSpec((1, DIM), lambda i: (0, 0)),
            pl.BlockSpec((DIM, OUT_DIM), lambda i: (0, 0)),
            pl.BlockSpec((1, OUT_DIM), lambda i: (0, 0)),
        ],
        out_specs=pl.BlockSpec((NB, K, OUT_DIM), lambda i: (i, 0, 0)),
        out_shape=jax.ShapeDtypeStruct((N, K, OUT_DIM), jnp.float32),
    )(tg, t, W3a_hi, Wnode, Wcat, b3a, W3b, b3b)


def kernel(xyz, knn, W1, b1, W2, b2, W3a, b3a, W3b, b3b):
    W3a_lo = W3a[:HALF, :]
    W3a_hi = W3a[HALF:, :]
    Wnode = jnp.concatenate([W3a_hi, W3a_lo], axis=0)
    Wcat = jnp.concatenate([jnp.zeros_like(W3a_lo), W3a_lo], axis=0)
    b1r = b1.reshape(1, HALF)
    b2r = b2.reshape(1, HALF)
    b3ar = b3a.reshape(1, DIM)
    b3br = b3b.reshape(1, OUT_DIM)

    t = _k1(xyz, W1, b1r, W2, b2r)

    knn_flat = knn.reshape(-1).astype(jnp.int32)
    knn_flat_pad = jnp.pad(knn_flat, (0, (NPAD - N) * K))
    tg_pad = _k2(knn_flat_pad, t)

    return _k3(tg_pad, t, W3a_hi, Wnode, Wcat, b3ar, W3b, b3br)
